# baseline (speedup 1.0000x reference)
import jax
import jax.numpy as jnp
from jax.experimental import pallas as pl

HIDDEN = 2048
EXPERTS = 16
TOPK = 8
HALF = 512


def _topk8(logits):
    sub = jax.lax.broadcasted_iota(jnp.int32, logits.shape, 0)
    vals = logits
    top_vals = []
    top_idx = []
    for _ in range(TOPK):
        m = jnp.max(vals, axis=0, keepdims=True)
        is_max = vals == m
        idx = jnp.min(jnp.where(is_max, sub, EXPERTS), axis=0, keepdims=True)
        top_vals.append(m)
        top_idx.append(idx)
        vals = jnp.where(sub == idx, -jnp.inf, vals)
    v = jnp.concatenate(top_vals, axis=0)
    e = jnp.exp(v - v[:1, :])
    return e / jnp.sum(e, axis=0, keepdims=True), jnp.concatenate(top_idx, axis=0)


def _gate_kernel(hs_a, hs_b, w_ref, w_out_ref, i_out_ref):
    for half, hs_ref in enumerate((hs_a, hs_b)):
        logits = jax.lax.dot_general(
            w_ref[...], hs_ref[...],
            dimension_numbers=(((1,), (1,)), ((), ())),
            preferred_element_type=jnp.float32,
        )
        w, i = _topk8(logits)
        w_out_ref[:, pl.ds(half * HALF, HALF)] = w
        i_out_ref[:, pl.ds(half * HALF, HALF)] = i


@jax.jit
def kernel(hidden_states, W):
    hs = hidden_states.reshape(-1, HIDDEN)
    n = hs.shape[0]
    grid = (n // (2 * HALF),)
    w_a, i_a = pl.pallas_call(
        _gate_kernel,
        grid=grid,
        in_specs=[
            pl.BlockSpec((HALF, HIDDEN), lambda i: (2 * i, 0)),
            pl.BlockSpec((HALF, HIDDEN), lambda i: (2 * i + 1, 0)),
            pl.BlockSpec((EXPERTS, HIDDEN), lambda i: (0, 0)),
        ],
        out_specs=[
            pl.BlockSpec((TOPK, 2 * HALF), lambda i: (0, i)),
            pl.BlockSpec((TOPK, 2 * HALF), lambda i: (0, i)),
        ],
        out_shape=[
            jax.ShapeDtypeStruct((TOPK, n), jnp.float32),
            jax.ShapeDtypeStruct((TOPK, n), jnp.int32),
        ],
    )(hs, hs, W)
    return (w_a.T, i_a.T)
